# baseline (speedup 1.0000x reference)
import functools

import jax
import jax.numpy as jnp
from jax import lax
from jax.experimental import pallas as pl
from jax.experimental.pallas import tpu as pltpu
from jax.experimental.pallas import tpu_sc as plsc

_BATCH = 4096
_K = 64
_NC = 2
_NS = 16
_NW = _NC * _NS
_BPW = _BATCH // _NW
_L = 16
_GROUPS = _BPW // _L
_G_B = 3.5

_mesh = plsc.VectorSubcoreMesh(core_axis_name="c", subcore_axis_name="s")


@functools.partial(
    pl.kernel,
    mesh=_mesh,
    out_type=jax.ShapeDtypeStruct((_BATCH,), jnp.float32),
    compiler_params=pltpu.CompilerParams(needs_layout_passes=False),
    scratch_types=[
        pltpu.VMEM((_BPW,), jnp.int32),
        pltpu.VMEM((_BPW,), jnp.int32),
        pltpu.VMEM((_BPW,), jnp.int32),
        pltpu.VMEM((_BPW,), jnp.int32),
        pltpu.VMEM((_BPW, 2 * _K), jnp.float32),
        pltpu.VMEM((_BPW, 2 * _K), jnp.float32),
        pltpu.VMEM((_BPW,), jnp.float32),
        pltpu.VMEM((_BPW,), jnp.float32),
        pltpu.VMEM((_BPW,), jnp.float32),
        pltpu.SemaphoreType.DMA,
    ],
)
def _mf_sc(uid_hbm, iid_hbm, user_m_hbm, item_m_hbm, user_b_hbm, item_b_hbm,
           out_hbm, uid_v, iid_v, urow_v, irow_v, urows, irows, ub_v, ib_v,
           out_v, sem):
    wid = lax.axis_index("s") * _NC + lax.axis_index("c")
    base = wid * _BPW
    pltpu.sync_copy(uid_hbm.at[pl.ds(base, _BPW)], uid_v)
    pltpu.sync_copy(iid_hbm.at[pl.ds(base, _BPW)], iid_v)
    for g in range(_GROUPS):
        s = pl.ds(g * _L, _L)
        urow_v[s] = lax.shift_right_logical(uid_v[s], 1)
        irow_v[s] = lax.shift_right_logical(iid_v[s], 1)
    c1 = pltpu.async_copy(user_m_hbm.at[urow_v], urows, sem)
    c2 = pltpu.async_copy(item_m_hbm.at[irow_v], irows, sem)
    c3 = pltpu.async_copy(user_b_hbm.at[uid_v], ub_v, sem)
    c4 = pltpu.async_copy(item_b_hbm.at[iid_v], ib_v, sem)
    c1.wait()
    c2.wait()
    c3.wait()
    c4.wait()
    lane = lax.iota(jnp.int32, _L)
    for g in range(_GROUPS):
        s = pl.ds(g * _L, _L)
        p_idx = lane + g * _L
        ucol0 = lax.shift_left(jnp.bitwise_and(uid_v[s], 1), 6)
        icol0 = lax.shift_left(jnp.bitwise_and(iid_v[s], 1), 6)
        acc = ub_v[s] + ib_v[s] + jnp.float32(_G_B)
        for d in range(_K):
            u = plsc.load_gather(urows, [p_idx, ucol0 + d])
            v = plsc.load_gather(irows, [p_idx, icol0 + d])
            acc = acc + u * v
        out_v[s] = acc
    pltpu.sync_copy(out_v, out_hbm.at[pl.ds(base, _BPW)])


def kernel(x, user_m, item_m, user_b, item_b):
    uid = x[:, 0]
    iid = x[:, 1]
    um2 = user_m.reshape(user_m.shape[0] // 2, 2 * _K)
    im2 = item_m.reshape(item_m.shape[0] // 2, 2 * _K)
    return _mf_sc(uid, iid, um2, im2, user_b, item_b)


# speedup vs baseline: 1.0058x; 1.0058x over previous
import functools

import jax
import jax.numpy as jnp
from jax import lax
from jax.experimental import pallas as pl
from jax.experimental.pallas import tpu as pltpu
from jax.experimental.pallas import tpu_sc as plsc

_BATCH = 4096
_K = 64
_NC = 2
_NS = 16
_NW = _NC * _NS
_BPW = _BATCH // _NW
_L = 16
_GROUPS = _BPW // _L
_G_B = 3.5

_mesh = plsc.VectorSubcoreMesh(core_axis_name="c", subcore_axis_name="s")


@functools.partial(
    pl.kernel,
    mesh=_mesh,
    out_type=jax.ShapeDtypeStruct((_BATCH,), jnp.float32),
    compiler_params=pltpu.CompilerParams(
        needs_layout_passes=False, use_tc_tiling_on_sc=False),
    scratch_types=[
        pltpu.VMEM((_BPW,), jnp.int32),
        pltpu.VMEM((_BPW,), jnp.int32),
        pltpu.VMEM((_BPW, _K), jnp.float32),
        pltpu.VMEM((_BPW, _K), jnp.float32),
        pltpu.VMEM((_BPW,), jnp.float32),
        pltpu.VMEM((_BPW,), jnp.float32),
        pltpu.VMEM((_BPW,), jnp.float32),
        pltpu.SemaphoreType.DMA,
    ],
)
def _mf_sc(uid_hbm, iid_hbm, user_m_hbm, item_m_hbm, user_b_hbm, item_b_hbm,
           out_hbm, uid_v, iid_v, urows, irows, ub_v, ib_v, out_v, sem):
    wid = lax.axis_index("s") * _NC + lax.axis_index("c")
    base = wid * _BPW
    pltpu.sync_copy(uid_hbm.at[pl.ds(base, _BPW)], uid_v)
    pltpu.sync_copy(iid_hbm.at[pl.ds(base, _BPW)], iid_v)
    c1 = pltpu.async_copy(user_m_hbm.at[uid_v], urows, sem)
    c2 = pltpu.async_copy(item_m_hbm.at[iid_v], irows, sem)
    c3 = pltpu.async_copy(user_b_hbm.at[uid_v], ub_v, sem)
    c4 = pltpu.async_copy(item_b_hbm.at[iid_v], ib_v, sem)
    c1.wait()
    c2.wait()
    c3.wait()
    c4.wait()
    lane = lax.iota(jnp.int32, _L)
    for g in range(_GROUPS):
        s = pl.ds(g * _L, _L)
        p_idx = lane + g * _L
        zero = jnp.zeros((_L,), jnp.int32)
        acc = ub_v[s] + ib_v[s] + jnp.float32(_G_B)
        for d in range(_K):
            u = plsc.load_gather(urows, [p_idx, zero + d])
            v = plsc.load_gather(irows, [p_idx, zero + d])
            acc = acc + u * v
        out_v[s] = acc
    pltpu.sync_copy(out_v, out_hbm.at[pl.ds(base, _BPW)])


def kernel(x, user_m, item_m, user_b, item_b):
    uid = x[:, 0]
    iid = x[:, 1]
    return _mf_sc(uid, iid, user_m, item_m, user_b, item_b)


# speedup vs baseline: 1.1557x; 1.1490x over previous
import functools

import jax
import jax.numpy as jnp
from jax import lax
from jax.experimental import pallas as pl
from jax.experimental.pallas import tpu as pltpu
from jax.experimental.pallas import tpu_sc as plsc

_BATCH = 4096
_K = 64
_NC = 2
_NS = 16
_NW = _NC * _NS
_BPW = _BATCH // _NW
_L = 16
_GROUPS = _BPW // _L
_G_B = 3.5

_mesh = plsc.VectorSubcoreMesh(core_axis_name="c", subcore_axis_name="s")


@functools.partial(
    pl.kernel,
    mesh=_mesh,
    out_type=jax.ShapeDtypeStruct((_BATCH,), jnp.float32),
    compiler_params=pltpu.CompilerParams(
        needs_layout_passes=False, use_tc_tiling_on_sc=True),
    scratch_types=[
        pltpu.VMEM((_BPW,), jnp.int32),
        pltpu.VMEM((_BPW,), jnp.int32),
        pltpu.VMEM((_BPW, 2 * _K), jnp.float32),
        pltpu.VMEM((_BPW, 2 * _K), jnp.float32),
        pltpu.VMEM((_BPW,), jnp.float32),
        pltpu.VMEM((_BPW,), jnp.float32),
        pltpu.VMEM((_BPW,), jnp.float32),
        pltpu.SemaphoreType.DMA,
    ],
)
def _mf_sc(uid_hbm, iid_hbm, mix_hbm, user_b_hbm, item_b_hbm,
           out_hbm, uid_v, iid_v, urows, irows, ub_v, ib_v, out_v, sem):
    wid = lax.axis_index("s") * _NC + lax.axis_index("c")
    base = wid * _BPW
    pltpu.sync_copy(uid_hbm.at[pl.ds(base, _BPW)], uid_v)
    pltpu.sync_copy(iid_hbm.at[pl.ds(base, _BPW)], iid_v)
    c1 = pltpu.async_copy(mix_hbm.at[uid_v], urows, sem)
    c2 = pltpu.async_copy(mix_hbm.at[iid_v], irows, sem)
    c3 = pltpu.async_copy(user_b_hbm.at[uid_v], ub_v, sem)
    c4 = pltpu.async_copy(item_b_hbm.at[iid_v], ib_v, sem)
    c1.wait()
    c2.wait()
    c3.wait()
    c4.wait()
    lane = lax.iota(jnp.int32, _L)
    for g in range(_GROUPS):
        s = pl.ds(g * _L, _L)
        p_idx = lane + g * _L
        zero = jnp.zeros((_L,), jnp.int32)
        acc = ub_v[s] + ib_v[s] + jnp.float32(_G_B)
        for d in range(_K):
            u = plsc.load_gather(urows, [p_idx, zero + d])
            v = plsc.load_gather(irows, [p_idx, zero + (_K + d)])
            acc = acc + u * v
        out_v[s] = acc
    pltpu.sync_copy(out_v, out_hbm.at[pl.ds(base, _BPW)])


def kernel(x, user_m, item_m, user_b, item_b):
    uid = x[:, 0]
    iid = x[:, 1]
    mix = jnp.concatenate([user_m, item_m], axis=1)
    return _mf_sc(uid, iid, mix, user_b, item_b)
